# baseline (speedup 1.0000x reference)
import functools

import jax
import jax.numpy as jnp
from jax import lax
from jax.experimental import pallas as pl
from jax.experimental.pallas import tpu as pltpu
from jax.experimental.pallas import tpu_sc as plsc

_IDX_CHUNK = 128
_VPAD = 56
_DPAD = 128


@functools.lru_cache(maxsize=None)
def _make_sc_gather(B):
    info = plsc.get_sparse_core_info()
    NC = info.num_cores
    NW = NC * info.num_subcores
    b_per_w = B // NW
    n_ch = b_per_w // _IDX_CHUNK
    mesh = plsc.VectorSubcoreMesh(core_axis_name="c", subcore_axis_name="s")

    @functools.partial(
        pl.kernel,
        out_type=jax.ShapeDtypeStruct((B, _DPAD), jnp.float32),
        mesh=mesh,
        scratch_types=[
            pltpu.VMEM((b_per_w,), jnp.int32),
            pltpu.VMEM((b_per_w, _DPAD), jnp.float32),
            pltpu.VMEM_SHARED((_VPAD, _DPAD), jnp.float32),
            pltpu.SemaphoreType.DMA,
            pltpu.SemaphoreType.DMA,
        ],
    )
    def gather_kernel(table_hbm, idx_hbm, out_hbm, idx_v, rows_v, tab_sp, sem, sem_o):
        sid = lax.axis_index("s")
        wid = sid * NC + lax.axis_index("c")
        base = wid * b_per_w

        @pl.when(sid == 0)
        def _():
            pltpu.sync_copy(table_hbm, tab_sp)

        pltpu.sync_copy(idx_hbm.at[pl.ds(base, b_per_w)], idx_v)
        plsc.subcore_barrier()
        copies = [
            pltpu.async_copy(
                tab_sp.at[idx_v.at[pl.ds(j * _IDX_CHUNK, _IDX_CHUNK)]],
                rows_v.at[pl.ds(j * _IDX_CHUNK, _IDX_CHUNK)],
                sem,
            )
            for j in range(n_ch)
        ]
        out_copies = []
        for j in range(n_ch):
            copies[j].wait()
            out_copies.append(
                pltpu.async_copy(
                    rows_v.at[pl.ds(j * _IDX_CHUNK, _IDX_CHUNK)],
                    out_hbm.at[pl.ds(base + j * _IDX_CHUNK, _IDX_CHUNK)],
                    sem_o,
                )
            )
        for c in out_copies:
            c.wait()

    return gather_kernel


@functools.lru_cache(maxsize=None)
def _make_tc_matmul(B, DX, DT, DO, blk):
    def body(x_ref, t_ref, w_ref, b_ref, o_ref):
        acc = jnp.dot(
            x_ref[...], w_ref[:DX, :], preferred_element_type=jnp.float32
        )
        acc += jnp.dot(
            t_ref[...][:, :DT], w_ref[DX:, :], preferred_element_type=jnp.float32
        )
        o_ref[...] = acc + b_ref[...]

    return pl.pallas_call(
        body,
        grid=(B // blk,),
        in_specs=[
            pl.BlockSpec((blk, DX), lambda i: (i, 0)),
            pl.BlockSpec((blk, _DPAD), lambda i: (i, 0)),
            pl.BlockSpec((DX + DT, DO), lambda i: (0, 0)),
            pl.BlockSpec((1, DO), lambda i: (0, 0)),
        ],
        out_specs=pl.BlockSpec((blk, DO), lambda i: (i, 0)),
        out_shape=jax.ShapeDtypeStruct((B, DO), jnp.float32),
    )


def kernel(x, time, time_embedding, W, b):
    B, DX = x.shape
    V, DT = time_embedding.shape
    DO = W.shape[1]
    tab_pad = jnp.pad(time_embedding, ((0, _VPAD - V), (0, _DPAD - DT)))
    t = _make_sc_gather(B)(tab_pad, time.astype(jnp.int32))
    return _make_tc_matmul(B, DX, DT, DO, 4096)(x, t, W, b.reshape(1, DO))


# speedup vs baseline: 1.0382x; 1.0382x over previous
import functools

import jax
import jax.numpy as jnp
from jax import lax
from jax.experimental import pallas as pl
from jax.experimental.pallas import tpu as pltpu
from jax.experimental.pallas import tpu_sc as plsc

_IDX_CHUNK = 128
_VPAD = 56
_DPAD = 128


@functools.lru_cache(maxsize=None)
def _make_sc_gather(B):
    info = plsc.get_sparse_core_info()
    NC = info.num_cores
    NW = NC * info.num_subcores
    b_per_w = B // NW
    n_ch = b_per_w // _IDX_CHUNK
    mesh = plsc.VectorSubcoreMesh(core_axis_name="c", subcore_axis_name="s")

    @functools.partial(
        pl.kernel,
        out_type=jax.ShapeDtypeStruct((B, _DPAD), jnp.float32),
        mesh=mesh,
        scratch_types=[
            pltpu.VMEM((b_per_w,), jnp.int32),
            pltpu.VMEM((b_per_w, _DPAD), jnp.float32),
            pltpu.VMEM_SHARED((_VPAD, _DPAD), jnp.float32),
            pltpu.SemaphoreType.DMA,
            pltpu.SemaphoreType.DMA,
        ],
    )
    def gather_kernel(table_hbm, idx_hbm, out_hbm, idx_v, rows_v, tab_sp, sem, sem_o):
        sid = lax.axis_index("s")
        wid = sid * NC + lax.axis_index("c")
        base = wid * b_per_w

        @pl.when(sid == 0)
        def _():
            pltpu.sync_copy(table_hbm, tab_sp)

        pltpu.sync_copy(idx_hbm.at[pl.ds(base, b_per_w)], idx_v)
        plsc.subcore_barrier()
        copies = [
            pltpu.async_copy(
                tab_sp.at[idx_v.at[pl.ds(j * _IDX_CHUNK, _IDX_CHUNK)]],
                rows_v.at[pl.ds(j * _IDX_CHUNK, _IDX_CHUNK)],
                sem,
            )
            for j in range(n_ch)
        ]
        out_copies = []
        for j in range(n_ch):
            copies[j].wait()
            out_copies.append(
                pltpu.async_copy(
                    rows_v.at[pl.ds(j * _IDX_CHUNK, _IDX_CHUNK)],
                    out_hbm.at[pl.ds(base + j * _IDX_CHUNK, _IDX_CHUNK)],
                    sem_o,
                )
            )
        for c in out_copies:
            c.wait()

    return gather_kernel


@functools.lru_cache(maxsize=None)
def _make_tc_matmul(B, DX, DT, DO, blk):
    def body(x_ref, t_ref, w_ref, b_ref, o_ref):
        acc = jnp.dot(
            x_ref[...], w_ref[:DX, :], preferred_element_type=jnp.float32
        )
        acc += jnp.dot(
            t_ref[...][:, :DT], w_ref[DX:, :], preferred_element_type=jnp.float32
        )
        o_ref[...] = acc + b_ref[...]

    return pl.pallas_call(
        body,
        grid=(B // blk,),
        in_specs=[
            pl.BlockSpec((blk, DX), lambda i: (i, 0)),
            pl.BlockSpec((blk, _DPAD), lambda i: (i, 0)),
            pl.BlockSpec((DX + DT, DO), lambda i: (0, 0)),
            pl.BlockSpec((1, DO), lambda i: (0, 0)),
        ],
        out_specs=pl.BlockSpec((blk, DO), lambda i: (i, 0)),
        out_shape=jax.ShapeDtypeStruct((B, DO), jnp.float32),
    )


def kernel(x, time, time_embedding, W, b):
    B, DX = x.shape
    V, DT = time_embedding.shape
    DO = W.shape[1]
    tab_pad = jnp.pad(time_embedding, ((0, _VPAD - V), (0, _DPAD - DT)))
    t = _make_sc_gather(B)(tab_pad, time.astype(jnp.int32))
    return _make_tc_matmul(B, DX, DT, DO, 8192)(x, t, W, b.reshape(1, DO))
